# baseline (speedup 1.0000x reference)
import functools

import jax
import jax.numpy as jnp
from jax import lax
from jax.experimental import pallas as pl
from jax.experimental.pallas import tpu as pltpu
from jax.experimental.pallas import tpu_sc as plsc

NC = 2
NS = 16
NW = NC * NS
LANES = 16
SUB = 8


def _sc_kernel(B, L, W_words, V_pad, HALF):
    cols_per_tile = B // NW
    n_groups = cols_per_tile // LANES
    n_blocks = L // SUB
    assert (n_blocks - 1) % 4 == 0

    mesh = plsc.VectorSubcoreMesh(
        core_axis_name="c", subcore_axis_name="s",
        num_cores=NC, num_subcores=NS)

    @functools.partial(
        pl.kernel,
        out_type=jax.ShapeDtypeStruct((B,), jnp.float32),
        mesh=mesh,
        scratch_types=[
            pltpu.VMEM((W_words,), jnp.int32),
            pltpu.VMEM((V_pad,), jnp.float32),
            pltpu.VMEM((4, SUB, 128), jnp.int32),
            pltpu.VMEM((cols_per_tile,), jnp.float32),
            pltpu.VMEM((LANES,), jnp.float32),
            pltpu.VMEM_SHARED((W_words,), jnp.int32),
            pltpu.VMEM_SHARED((V_pad,), jnp.float32),
            pltpu.SemaphoreType.DMA,
            pltpu.SemaphoreType.DMA,
            pltpu.SemaphoreType.DMA,
            pltpu.SemaphoreType.DMA,
            pltpu.SemaphoreType.DMA,
        ],
        compiler_params=pltpu.CompilerParams(needs_layout_passes=False),
    )
    def body(g2l_hbm, emb_hbm, rules_hbm, bias_hbm, out_hbm,
             g2l_v, emb_v, rules_c, out_v, bias_v, g2l_sh, emb_sh,
             sem, s0, s1, s2, s3):
        sems = (s0, s1, s2, s3)
        sid = lax.axis_index("s")
        wid = sid * NC + lax.axis_index("c")
        col0 = wid * cols_per_tile

        @pl.when(sid == 0)
        def _():
            pltpu.async_copy(g2l_hbm, g2l_sh, sem).wait()

        @pl.when(sid == 1)
        def _():
            pltpu.async_copy(emb_hbm, emb_sh, sem).wait()

        c4 = pltpu.async_copy(bias_hbm, bias_v, sem)
        plsc.subcore_barrier()
        c1 = pltpu.async_copy(g2l_sh, g2l_v, sem)
        c2 = pltpu.async_copy(emb_sh, emb_v, sem)

        def fetch(blk, buf, s):
            return pltpu.async_copy(
                rules_hbm.at[pl.ds(blk * SUB, SUB), pl.ds(col0, 128)],
                rules_c.at[buf], s)

        fetch(0, 0, sems[0])
        fetch(1, 1, sems[1])
        fetch(2, 2, sems[2])

        c1.wait()
        c2.wait()
        c4.wait()

        def sq16(r):
            in_hi = r >= HALF
            word_idx = jnp.where(in_hi, r - HALF, r)
            w = plsc.load_gather(g2l_v, [word_idx])
            hi = jnp.bitwise_and(jnp.right_shift(w, 16), 0xFFFF)
            lo = jnp.bitwise_and(w, 0xFFFF)
            local = jnp.where(in_hi, hi, lo)
            v = plsc.load_gather(emb_v, [local])
            return v * v

        def block_acc(buf, accs):
            ref = rules_c.at[buf]

            def lbody(l, a):
                return tuple(
                    a[g] + sq16(ref[l, pl.ds(g * LANES, LANES)])
                    for g in range(n_groups))

            return lax.fori_loop(0, SUB, lbody, accs)

        def drain(blk, buf, s):
            pltpu.make_async_copy(
                rules_hbm.at[pl.ds(blk * SUB, SUB), pl.ds(col0, 128)],
                rules_c.at[buf], s).wait()

        def quad(i, accs):
            base = 4 * i
            for k in range(4):
                drain(base + k, k, sems[k])

                @pl.when(base + k + 3 < n_blocks)
                def _(k=k):
                    fetch(base + k + 3, (k + 3) % 4, sems[(k + 3) % 4])

                accs = block_acc(k, accs)
            return accs

        zeros = tuple(jnp.zeros((LANES,), jnp.float32) for _ in range(n_groups))
        accs = lax.fori_loop(0, (n_blocks - 1) // 4, quad, zeros)

        drain(n_blocks - 1, 0, sems[0])
        accs = block_acc(0, accs)

        bias_vec = bias_v[...]
        for g in range(n_groups):
            out_v[pl.ds(g * LANES, LANES)] = accs[g] + bias_vec
        pltpu.sync_copy(out_v, out_hbm.at[pl.ds(col0, cols_per_tile)])

    return body


def kernel(rules, global_to_local, emb_weight, bias):
    B, L = rules.shape
    V = emb_weight.shape[0]
    G = global_to_local.shape[0]

    gp = global_to_local.astype(jnp.int32)
    half = (G + 1) // 2
    packed = jnp.bitwise_or(gp[:half], jnp.left_shift(gp[half:2 * half], 16))
    W_words = (half + 15) // 16 * 16
    packed = jnp.pad(packed, (0, W_words - half))

    V_pad = (V + 15) // 16 * 16
    emb_p = jnp.pad(emb_weight.reshape(-1), (0, V_pad - V))

    bias_vec = jnp.broadcast_to(bias.reshape(()), (LANES,)).astype(jnp.float32)
    rules_t = rules.astype(jnp.int32).T

    out = _sc_kernel(B, L, W_words, V_pad, half)(packed, emb_p, rules_t, bias_vec)
    return out.reshape(B, 1)
